# Initial kernel scaffold; baseline (speedup 1.0000x reference)
import jax
import jax.numpy as jnp
from jax.experimental import pallas as pl


def kernel(inputs, W_ih, W_hh, b_ih, b_hh):
    raise NotImplementedError("write your pallas kernel here")


# speedup vs baseline: 10.9012x; 10.9012x over previous
import functools

import jax
import jax.numpy as jnp
from jax.experimental import pallas as pl
from jax.experimental.pallas import tpu as pltpu

_Z_CELL = 0.1
_Z_HID = 0.1
_TS = 8


def _proj_kernel(x_ref, w_ref, b_ref, o_ref):
    o_ref[...] = (
        jnp.dot(x_ref[...], w_ref[...], preferred_element_type=jnp.float32)
        + b_ref[...]
    )


def _lstm_kernel(xw_ref, whh_ref, o_ref, h_ref, c_ref, *, ts, hid):
    d = pl.program_id(0)

    @pl.when(pl.program_id(1) == 0)
    def _():
        h_ref[...] = jnp.zeros_like(h_ref)
        c_ref[...] = jnp.zeros_like(c_ref)

    for j in range(ts):
        jj = jnp.where(d == 0, j, ts - 1 - j)
        h = h_ref[...]
        c = c_ref[...]
        gates = xw_ref[jj] + jnp.dot(
            h, whh_ref[...], preferred_element_type=jnp.float32
        )
        gi = jax.nn.sigmoid(gates[:, :hid])
        gf = jax.nn.sigmoid(gates[:, hid : 2 * hid])
        gg = jnp.tanh(gates[:, 2 * hid : 3 * hid])
        go = jax.nn.sigmoid(gates[:, 3 * hid :])
        c_new = gf * c + gi * gg
        h_new = go * jnp.tanh(c_new)
        c_ref[...] = (1.0 - _Z_CELL) * c_new + _Z_CELL * c
        h_out = (1.0 - _Z_HID) * h_new + _Z_HID * h
        h_ref[...] = h_out
        o_ref[0, jj] = h_out


def kernel(inputs, W_ih, W_hh, b_ih, b_hh):
    T, B, I = inputs.shape
    G = W_ih.shape[0]
    hid = G // 4
    ts = _TS
    nt = T // ts

    x2d = inputs.reshape(T * B, I)
    bias = (b_ih + b_hh).reshape(1, G)
    bm = 1024
    xw = pl.pallas_call(
        _proj_kernel,
        grid=((T * B) // bm,),
        in_specs=[
            pl.BlockSpec((bm, I), lambda m: (m, 0)),
            pl.BlockSpec((I, G), lambda m: (0, 0)),
            pl.BlockSpec((1, G), lambda m: (0, 0)),
        ],
        out_specs=pl.BlockSpec((bm, G), lambda m: (m, 0)),
        out_shape=jax.ShapeDtypeStruct((T * B, G), jnp.float32),
        compiler_params=pltpu.CompilerParams(
            dimension_semantics=("parallel",),
            vmem_limit_bytes=48 * 1024 * 1024,
        ),
        name="lstm_in_proj",
    )(x2d, W_ih.T, bias).reshape(T, B, G)

    out2 = pl.pallas_call(
        functools.partial(_lstm_kernel, ts=ts, hid=hid),
        grid=(2, nt),
        in_specs=[
            pl.BlockSpec(
                (ts, B, G),
                lambda d, t: (jnp.where(d == 0, t, nt - 1 - t), 0, 0),
            ),
            pl.BlockSpec((hid, G), lambda d, t: (0, 0)),
        ],
        out_specs=pl.BlockSpec(
            (1, ts, B, hid),
            lambda d, t: (d, jnp.where(d == 0, t, nt - 1 - t), 0, 0),
        ),
        out_shape=jax.ShapeDtypeStruct((2, T, B, hid), jnp.float32),
        scratch_shapes=[
            pltpu.VMEM((B, hid), jnp.float32),
            pltpu.VMEM((B, hid), jnp.float32),
        ],
        compiler_params=pltpu.CompilerParams(
            dimension_semantics=("parallel", "arbitrary"),
            vmem_limit_bytes=48 * 1024 * 1024,
        ),
        name="lstm_recurrence",
    )(xw, W_hh.T)

    return out2[0] + out2[1]


# speedup vs baseline: 17.4107x; 1.5971x over previous
import functools

import jax
import jax.numpy as jnp
from jax.experimental import pallas as pl
from jax.experimental.pallas import tpu as pltpu

_Z_CELL = 0.1
_Z_HID = 0.1
_TS = 8


def _proj_kernel(x_ref, w_ref, b_ref, o_ref):
    o_ref[...] = (
        jnp.dot(x_ref[...], w_ref[...], preferred_element_type=jnp.float32)
        + b_ref[...]
    )


def _cell(xw, h, c, hid):
    gi = jax.nn.sigmoid(xw[:, :hid])
    gf = jax.nn.sigmoid(xw[:, hid : 2 * hid])
    gg = jnp.tanh(xw[:, 2 * hid : 3 * hid])
    go = jax.nn.sigmoid(xw[:, 3 * hid :])
    c_new = gf * c + gi * gg
    h_new = go * jnp.tanh(c_new)
    c_out = (1.0 - _Z_CELL) * c_new + _Z_CELL * c
    h_out = (1.0 - _Z_HID) * h_new + _Z_HID * h
    return h_out, c_out


def _lstm_kernel(xwf_ref, xwb_ref, whh_ref, of_ref, ob_ref, h_ref, c_ref, *, ts, hid, nb):
    @pl.when(pl.program_id(0) == 0)
    def _():
        h_ref[...] = jnp.zeros_like(h_ref)
        c_ref[...] = jnp.zeros_like(c_ref)

    for j in range(ts):
        jb = ts - 1 - j
        h = h_ref[...]
        gates = jnp.dot(
            h.astype(jnp.bfloat16), whh_ref[...], preferred_element_type=jnp.float32
        )
        hf, cf = _cell(gates[:nb] + xwf_ref[j], h[:nb], c_ref[:nb], hid)
        hb, cb = _cell(gates[nb:] + xwb_ref[jb], h[nb:], c_ref[nb:], hid)
        h_ref[:nb] = hf
        h_ref[nb:] = hb
        c_ref[:nb] = cf
        c_ref[nb:] = cb
        of_ref[j] = hf
        ob_ref[jb] = hb


def kernel(inputs, W_ih, W_hh, b_ih, b_hh):
    T, B, I = inputs.shape
    G = W_ih.shape[0]
    hid = G // 4
    ts = _TS
    nt = T // ts

    x2d = inputs.reshape(T * B, I)
    bias = (b_ih + b_hh).reshape(1, G)
    bm = 1024
    xw = pl.pallas_call(
        _proj_kernel,
        grid=((T * B) // bm,),
        in_specs=[
            pl.BlockSpec((bm, I), lambda m: (m, 0)),
            pl.BlockSpec((I, G), lambda m: (0, 0)),
            pl.BlockSpec((1, G), lambda m: (0, 0)),
        ],
        out_specs=pl.BlockSpec((bm, G), lambda m: (m, 0)),
        out_shape=jax.ShapeDtypeStruct((T * B, G), jnp.float32),
        compiler_params=pltpu.CompilerParams(
            dimension_semantics=("arbitrary",),
            vmem_limit_bytes=48 * 1024 * 1024,
        ),
        name="lstm_in_proj",
    )(x2d, W_ih.T, bias).reshape(T, B, G)

    out_f, out_b = pl.pallas_call(
        functools.partial(_lstm_kernel, ts=ts, hid=hid, nb=B),
        grid=(nt,),
        in_specs=[
            pl.BlockSpec((ts, B, G), lambda t: (t, 0, 0)),
            pl.BlockSpec((ts, B, G), lambda t: (nt - 1 - t, 0, 0)),
            pl.BlockSpec((hid, G), lambda t: (0, 0)),
        ],
        out_specs=[
            pl.BlockSpec((ts, B, hid), lambda t: (t, 0, 0)),
            pl.BlockSpec((ts, B, hid), lambda t: (nt - 1 - t, 0, 0)),
        ],
        out_shape=[
            jax.ShapeDtypeStruct((T, B, hid), jnp.float32),
            jax.ShapeDtypeStruct((T, B, hid), jnp.float32),
        ],
        scratch_shapes=[
            pltpu.VMEM((2 * B, hid), jnp.float32),
            pltpu.VMEM((2 * B, hid), jnp.float32),
        ],
        compiler_params=pltpu.CompilerParams(
            dimension_semantics=("arbitrary",),
            vmem_limit_bytes=48 * 1024 * 1024,
        ),
        name="lstm_recurrence",
    )(xw, xw, W_hh.T.astype(jnp.bfloat16))

    return out_f + out_b


# speedup vs baseline: 17.5306x; 1.0069x over previous
import functools

import jax
import jax.numpy as jnp
from jax.experimental import pallas as pl
from jax.experimental.pallas import tpu as pltpu

_Z_CELL = 0.1
_Z_HID = 0.1
_TS = 16


def _proj_kernel(x_ref, w_ref, b_ref, o_ref):
    o_ref[...] = (
        jnp.dot(x_ref[...], w_ref[...], preferred_element_type=jnp.float32)
        + b_ref[...]
    )


def _cell(xw, h, c, hid):
    gi = jax.nn.sigmoid(xw[:, :hid])
    gf = jax.nn.sigmoid(xw[:, hid : 2 * hid])
    gg = jnp.tanh(xw[:, 2 * hid : 3 * hid])
    go = jax.nn.sigmoid(xw[:, 3 * hid :])
    c_new = gf * c + gi * gg
    h_new = go * jnp.tanh(c_new)
    c_out = (1.0 - _Z_CELL) * c_new + _Z_CELL * c
    h_out = (1.0 - _Z_HID) * h_new + _Z_HID * h
    return h_out, c_out


def _lstm_kernel(xwf_ref, xwb_ref, whh_ref, of_ref, ob_ref, h_ref, c_ref, *, ts, hid, nb):
    @pl.when(pl.program_id(0) == 0)
    def _():
        h_ref[...] = jnp.zeros_like(h_ref)
        c_ref[...] = jnp.zeros_like(c_ref)

    for j in range(ts):
        jb = ts - 1 - j
        h = h_ref[...]
        gates = jnp.dot(
            h.astype(jnp.bfloat16), whh_ref[...], preferred_element_type=jnp.float32
        )
        hf, cf = _cell(gates[:nb] + xwf_ref[j], h[:nb], c_ref[:nb], hid)
        hb, cb = _cell(gates[nb:] + xwb_ref[jb], h[nb:], c_ref[nb:], hid)
        h_ref[:nb] = hf
        h_ref[nb:] = hb
        c_ref[:nb] = cf
        c_ref[nb:] = cb
        of_ref[j] = hf
        ob_ref[jb] = hb


def kernel(inputs, W_ih, W_hh, b_ih, b_hh):
    T, B, I = inputs.shape
    G = W_ih.shape[0]
    hid = G // 4
    ts = _TS
    nt = T // ts

    x2d = inputs.reshape(T * B, I)
    bias = (b_ih + b_hh).reshape(1, G)
    bm = 1024
    xw = pl.pallas_call(
        _proj_kernel,
        grid=((T * B) // bm,),
        in_specs=[
            pl.BlockSpec((bm, I), lambda m: (m, 0)),
            pl.BlockSpec((I, G), lambda m: (0, 0)),
            pl.BlockSpec((1, G), lambda m: (0, 0)),
        ],
        out_specs=pl.BlockSpec((bm, G), lambda m: (m, 0)),
        out_shape=jax.ShapeDtypeStruct((T * B, G), jnp.float32),
        compiler_params=pltpu.CompilerParams(
            dimension_semantics=("arbitrary",),
            vmem_limit_bytes=48 * 1024 * 1024,
        ),
        name="lstm_in_proj",
    )(x2d, W_ih.T, bias).reshape(T, B, G)

    out_f, out_b = pl.pallas_call(
        functools.partial(_lstm_kernel, ts=ts, hid=hid, nb=B),
        grid=(nt,),
        in_specs=[
            pl.BlockSpec((ts, B, G), lambda t: (t, 0, 0)),
            pl.BlockSpec((ts, B, G), lambda t: (nt - 1 - t, 0, 0)),
            pl.BlockSpec((hid, G), lambda t: (0, 0)),
        ],
        out_specs=[
            pl.BlockSpec((ts, B, hid), lambda t: (t, 0, 0)),
            pl.BlockSpec((ts, B, hid), lambda t: (nt - 1 - t, 0, 0)),
        ],
        out_shape=[
            jax.ShapeDtypeStruct((T, B, hid), jnp.float32),
            jax.ShapeDtypeStruct((T, B, hid), jnp.float32),
        ],
        scratch_shapes=[
            pltpu.VMEM((2 * B, hid), jnp.float32),
            pltpu.VMEM((2 * B, hid), jnp.float32),
        ],
        compiler_params=pltpu.CompilerParams(
            dimension_semantics=("arbitrary",),
            vmem_limit_bytes=48 * 1024 * 1024,
        ),
        name="lstm_recurrence",
    )(xw, xw, W_hh.T.astype(jnp.bfloat16))

    return out_f + out_b
